# baseline (speedup 1.0000x reference)
import functools

import jax
import jax.numpy as jnp
from jax.experimental import pallas as pl
from jax.experimental.pallas import tpu as pltpu

_K = 1024
_D = 64
_COMMITMENT_COST = 0.25


def _vq_block_kernel(x_ref, emb_ref, embt_ref,
                     q_ref, idx_ref, cnt_ref, loss_ref, ppl_ref,
                     b_ref,
                     *, n_rows: int):
    i = pl.program_id(0)
    nb = pl.num_programs(0)

    xb = x_ref[...]
    emb = emb_ref[...]
    embt = embt_ref[...]

    @pl.when(i == 0)
    def _precompute():
        b_ref[...] = jnp.sum(embt * embt, axis=0, keepdims=True)

    a = jnp.sum(xb * xb, axis=1, keepdims=True)
    b = b_ref[...]
    mm = jax.lax.dot_general(
        xb, embt, (((1,), (0,)), ((), ())),
        preferred_element_type=jnp.float32)
    dist = (a + b) - 2.0 * mm

    m = jnp.min(dist, axis=1, keepdims=True)
    colids = jax.lax.broadcasted_iota(jnp.int32, dist.shape, 1)
    idx = jnp.min(jnp.where(dist == m, colids, _K), axis=1)
    idx_ref[...] = idx[:, None]

    onehot = (colids == idx[:, None]).astype(jnp.float32)
    q = jax.lax.dot_general(
        onehot, emb, (((1,), (0,)), ((), ())),
        preferred_element_type=jnp.float32)
    q_ref[...] = q

    diff = q - xb
    part_loss = jnp.sum(diff * diff).reshape(1, 1)
    part_cnt = jnp.sum(onehot, axis=0, keepdims=True)

    @pl.when(i == 0)
    def _init():
        loss_ref[...] = jnp.zeros_like(loss_ref)
        cnt_ref[...] = jnp.zeros_like(cnt_ref)
        ppl_ref[...] = jnp.zeros_like(ppl_ref)

    loss_ref[...] += part_loss
    cnt_ref[...] += part_cnt

    @pl.when(i == nb - 1)
    def _finalize():
        mse = loss_ref[...] / (n_rows * _D)
        loss_ref[...] = mse + _COMMITMENT_COST * mse
        probs = cnt_ref[...] / float(n_rows)
        avg = jnp.sum(probs, axis=1, keepdims=True) / _K
        ppl_ref[...] = jnp.exp(-(avg * jnp.log(avg + 1e-10)))


def kernel(x, emb_weight):
    n_rows = x.shape[0] * x.shape[1]
    flat = x.reshape(n_rows, _D)
    blk = 576
    nb = n_rows // blk

    q, idx, _cnt, loss, ppl = pl.pallas_call(
        functools.partial(_vq_block_kernel, n_rows=n_rows),
        grid=(nb,),
        in_specs=[
            pl.BlockSpec((blk, _D), lambda i: (i, 0)),
            pl.BlockSpec((_K, _D), lambda i: (0, 0)),
            pl.BlockSpec((_D, _K), lambda i: (0, 0)),
        ],
        out_specs=[
            pl.BlockSpec((blk, _D), lambda i: (i, 0)),
            pl.BlockSpec((blk, 1), lambda i: (i, 0)),
            pl.BlockSpec((1, _K), lambda i: (0, 0)),
            pl.BlockSpec((1, 1), lambda i: (0, 0)),
            pl.BlockSpec((1, 1), lambda i: (0, 0)),
        ],
        out_shape=[
            jax.ShapeDtypeStruct((n_rows, _D), jnp.float32),
            jax.ShapeDtypeStruct((n_rows, 1), jnp.int32),
            jax.ShapeDtypeStruct((1, _K), jnp.float32),
            jax.ShapeDtypeStruct((1, 1), jnp.float32),
            jax.ShapeDtypeStruct((1, 1), jnp.float32),
        ],
        scratch_shapes=[pltpu.VMEM((1, _K), jnp.float32)],
    )(flat, emb_weight, emb_weight.T)

    return (q.reshape(x.shape), loss[0, 0], ppl[0, 0], idx)


# speedup vs baseline: 1.1475x; 1.1475x over previous
import functools

import jax
import jax.numpy as jnp
from jax.experimental import pallas as pl
from jax.experimental.pallas import tpu as pltpu

_K = 1024
_D = 64
_COMMITMENT_COST = 0.25


def _vq_block_kernel(x_ref, emb_ref, embt_ref,
                     q_ref, idx_ref, cnt_ref, loss_ref, ppl_ref,
                     b_ref,
                     *, n_rows: int):
    i = pl.program_id(0)
    nb = pl.num_programs(0)

    xb = x_ref[...]
    emb = emb_ref[...]
    embt = embt_ref[...]

    @pl.when(i == 0)
    def _precompute():
        b_ref[...] = jnp.sum(embt * embt, axis=0, keepdims=True)

    a = jnp.sum(xb * xb, axis=1, keepdims=True)
    b = b_ref[...]
    mm = jax.lax.dot_general(
        xb, embt, (((1,), (0,)), ((), ())),
        preferred_element_type=jnp.float32)
    dist = (a + b) - 2.0 * mm

    m = jnp.min(dist, axis=1, keepdims=True)
    colids = jax.lax.broadcasted_iota(jnp.int32, dist.shape, 1)
    idx = jnp.min(jnp.where(dist == m, colids, _K), axis=1)
    idx_ref[...] = idx[:, None]

    onehot = (colids == idx[:, None]).astype(jnp.float32)
    q = jax.lax.dot_general(
        onehot, emb, (((1,), (0,)), ((), ())),
        preferred_element_type=jnp.float32)
    q_ref[...] = q

    diff = q - xb
    part_loss = jnp.sum(diff * diff).reshape(1, 1)
    part_cnt = jnp.sum(onehot, axis=0, keepdims=True)

    @pl.when(i == 0)
    def _init():
        loss_ref[...] = jnp.zeros_like(loss_ref)
        cnt_ref[...] = jnp.zeros_like(cnt_ref)
        ppl_ref[...] = jnp.zeros_like(ppl_ref)

    loss_ref[...] += part_loss
    cnt_ref[...] += part_cnt

    @pl.when(i == nb - 1)
    def _finalize():
        mse = loss_ref[...] / (n_rows * _D)
        loss_ref[...] = mse + _COMMITMENT_COST * mse
        probs = cnt_ref[...] / float(n_rows)
        avg = jnp.sum(probs, axis=1, keepdims=True) / _K
        ppl_ref[...] = jnp.exp(-(avg * jnp.log(avg + 1e-10)))


def kernel(x, emb_weight):
    n_rows = x.shape[0] * x.shape[1]
    flat = x.reshape(n_rows, _D)
    blk = 4608
    nb = n_rows // blk

    q, idx, _cnt, loss, ppl = pl.pallas_call(
        functools.partial(_vq_block_kernel, n_rows=n_rows),
        grid=(nb,),
        in_specs=[
            pl.BlockSpec((blk, _D), lambda i: (i, 0)),
            pl.BlockSpec((_K, _D), lambda i: (0, 0)),
            pl.BlockSpec((_D, _K), lambda i: (0, 0)),
        ],
        out_specs=[
            pl.BlockSpec((blk, _D), lambda i: (i, 0)),
            pl.BlockSpec((blk, 1), lambda i: (i, 0)),
            pl.BlockSpec((1, _K), lambda i: (0, 0)),
            pl.BlockSpec((1, 1), lambda i: (0, 0)),
            pl.BlockSpec((1, 1), lambda i: (0, 0)),
        ],
        out_shape=[
            jax.ShapeDtypeStruct((n_rows, _D), jnp.float32),
            jax.ShapeDtypeStruct((n_rows, 1), jnp.int32),
            jax.ShapeDtypeStruct((1, _K), jnp.float32),
            jax.ShapeDtypeStruct((1, 1), jnp.float32),
            jax.ShapeDtypeStruct((1, 1), jnp.float32),
        ],
        scratch_shapes=[pltpu.VMEM((1, _K), jnp.float32)],
    )(flat, emb_weight, emb_weight.T)

    return (q.reshape(x.shape), loss[0, 0], ppl[0, 0], idx)
